# Initial kernel scaffold; baseline (speedup 1.0000x reference)
import jax
import jax.numpy as jnp
from jax.experimental import pallas as pl


def kernel(input, targets):
    raise NotImplementedError("write your pallas kernel here")


# speedup vs baseline: 15.2563x; 15.2563x over previous
import functools

import jax
import jax.numpy as jnp
from jax.experimental import pallas as pl
from jax.experimental.pallas import tpu as pltpu

LO = 1e-5
SKIP = -5.0


def _ctc_kernel(in_ref, cls_ref, out_ref, lm_ref, v_ref, *, s_chunk, t_pad,
                n_sc, n_valid_t):
    g = pl.program_id(0)
    sc = pl.program_id(1)
    del g
    G = in_ref.shape[1]
    C = in_ref.shape[2]

    x = in_ref[...]
    xc = jnp.maximum(x, LO)
    ssum = jnp.sum(xc, axis=2, keepdims=True)
    p = (xc / ssum).astype(jnp.bfloat16)
    iota_c = jax.lax.broadcasted_iota(jnp.int32, (C, t_pad), 0)
    for n in range(G):
        cls_row = cls_ref[n, :].reshape(1, t_pad)
        oh = jnp.where(iota_c == cls_row, 1.0, 0.0).astype(jnp.bfloat16)
        pn = p[:, n, :].reshape(s_chunk, C)
        res = jnp.dot(pn, oh, preferred_element_type=jnp.float32)
        lm_ref[:, n, :] = jnp.log(res)

    @pl.when(sc == 0)
    def _():
        lane = jax.lax.broadcasted_iota(jnp.int32, (G, t_pad), 1)
        v_ref[...] = SKIP * lane.astype(jnp.float32)

    lane_i = jax.lax.broadcasted_iota(jnp.int32, (G, t_pad), 1)
    base = (sc * s_chunk).astype(jnp.float32)

    def body(s, v):
        m = lm_ref[s]
        w = pltpu.roll(v, 1, axis=1)
        si = base + s.astype(jnp.float32)
        w = jnp.where(lane_i == 0, SKIP * si, w)
        xx = v + m
        yy = w + m
        d = xx - yy
        return jnp.where(jnp.abs(d) > 10.0,
                         jnp.maximum(xx, yy),
                         jnp.log(jnp.exp(jnp.clip(d, -20.0, 20.0)) + 1.0) + yy)

    v = jax.lax.fori_loop(0, s_chunk, body, v_ref[...])
    v_ref[...] = v

    @pl.when(sc == n_sc - 1)
    def _():
        x1 = v[:, n_valid_t - 1:n_valid_t]
        y1 = v[:, n_valid_t - 2:n_valid_t - 1]
        d = x1 - y1
        la = jnp.where(jnp.abs(d) > 10.0,
                       jnp.maximum(x1, y1),
                       jnp.log(jnp.exp(jnp.clip(d, -20.0, 20.0)) + 1.0) + y1)
        out_ref[...] = jnp.broadcast_to(-la, (G, 128)).reshape(1, G, 128)


@jax.jit
def kernel(input, targets):
    S, N, C = input.shape
    L = targets.shape[0]
    T = 2 * L + 1
    t_pad = 512
    G = N // 2
    s_chunk = 256
    n_sc = S // s_chunk

    tgt = targets.astype(jnp.int32)
    cls = jnp.zeros((N, t_pad), jnp.int32)
    cls = cls.at[:, 1:2 * L:2].set(tgt.T)

    out = pl.pallas_call(
        functools.partial(_ctc_kernel, s_chunk=s_chunk, t_pad=t_pad,
                          n_sc=n_sc, n_valid_t=T),
        grid=(2, n_sc),
        in_specs=[
            pl.BlockSpec((s_chunk, G, C), lambda g, sc: (sc, g, 0)),
            pl.BlockSpec((G, t_pad), lambda g, sc: (g, 0)),
        ],
        out_specs=pl.BlockSpec((1, G, 128), lambda g, sc: (g, 0, 0)),
        out_shape=jax.ShapeDtypeStruct((2, G, 128), jnp.float32),
        scratch_shapes=[
            pltpu.VMEM((s_chunk, G, t_pad), jnp.float32),
            pltpu.VMEM((G, t_pad), jnp.float32),
        ],
        compiler_params=pltpu.CompilerParams(
            dimension_semantics=("parallel", "arbitrary"),
            vmem_limit_bytes=100 * 1024 * 1024,
        ),
    )(input, cls)

    losses = out.reshape(N, 128)[:, 0]
    return jnp.sum(losses) / N


# speedup vs baseline: 16.3834x; 1.0739x over previous
import functools

import jax
import jax.numpy as jnp
from jax.experimental import pallas as pl
from jax.experimental.pallas import tpu as pltpu

LO = 1e-5
SKIP = -5.0


def _ctc_kernel(in_ref, cls_ref, out_ref, lm_ref, v_ref, *, s_chunk, t_pad,
                n_sc, n_valid_t):
    g = pl.program_id(0)
    sc = pl.program_id(1)
    del g
    G = in_ref.shape[1]
    C = in_ref.shape[2]

    x = in_ref[...]
    xc = jnp.maximum(x, LO)
    ssum = jnp.sum(xc, axis=2, keepdims=True)
    p = (xc / ssum).astype(jnp.bfloat16)
    iota_c = jax.lax.broadcasted_iota(jnp.int32, (C, t_pad), 0)
    for n in range(G):
        cls_row = cls_ref[n, :].reshape(1, t_pad)
        oh = jnp.where(iota_c == cls_row, 1.0, 0.0).astype(jnp.bfloat16)
        pn = p[:, n, :].reshape(s_chunk, C)
        res = jnp.dot(pn, oh, preferred_element_type=jnp.float32)
        lm_ref[:, n, :] = jnp.log(res)

    @pl.when(sc == 0)
    def _():
        lane = jax.lax.broadcasted_iota(jnp.int32, (G, t_pad), 1)
        t_of_lane = 4 * (lane & 127) + (lane >> 7)
        v_ref[...] = SKIP * t_of_lane.astype(jnp.float32)

    lane_i = jax.lax.broadcasted_iota(jnp.int32, (G, 128), 1)
    base = (sc * s_chunk).astype(jnp.float32)

    def log_add_m(v, w, m):
        xx = v + m
        yy = w + m
        d = xx - yy
        return jnp.where(jnp.abs(d) > 10.0,
                         jnp.maximum(xx, yy),
                         jnp.log(jnp.exp(jnp.clip(d, -20.0, 20.0)) + 1.0) + yy)

    def body(s, carry):
        v0, v1, v2, v3 = carry
        m = lm_ref[s]
        si = base + s.astype(jnp.float32)
        w0 = pltpu.roll(v3, 1, axis=1)
        w0 = jnp.where(lane_i == 0, SKIP * si, w0)
        n0 = log_add_m(v0, w0, m[:, 0:128])
        n1 = log_add_m(v1, v0, m[:, 128:256])
        n2 = log_add_m(v2, v1, m[:, 256:384])
        n3 = log_add_m(v3, v2, m[:, 384:512])
        return n0, n1, n2, n3

    carry0 = (v_ref[:, 0:128], v_ref[:, 128:256],
              v_ref[:, 256:384], v_ref[:, 384:512])
    v0, v1, v2, v3 = jax.lax.fori_loop(0, s_chunk, body, carry0)
    v_ref[:, 0:128] = v0
    v_ref[:, 128:256] = v1
    v_ref[:, 256:384] = v2
    v_ref[:, 384:512] = v3

    @pl.when(sc == n_sc - 1)
    def _():
        vs = (v0, v1, v2, v3)
        t1, t2 = n_valid_t - 1, n_valid_t - 2
        x1 = vs[t1 % 4][:, t1 // 4:t1 // 4 + 1]
        y1 = vs[t2 % 4][:, t2 // 4:t2 // 4 + 1]
        d = x1 - y1
        la = jnp.where(jnp.abs(d) > 10.0,
                       jnp.maximum(x1, y1),
                       jnp.log(jnp.exp(jnp.clip(d, -20.0, 20.0)) + 1.0) + y1)
        out_ref[...] = jnp.broadcast_to(-la, (G, 128)).reshape(1, G, 128)


@jax.jit
def kernel(input, targets):
    S, N, C = input.shape
    L = targets.shape[0]
    T = 2 * L + 1
    t_pad = 512
    G = N // 2
    s_chunk = 256
    n_sc = S // s_chunk

    tgt = targets.astype(jnp.int32)
    cls = jnp.zeros((N, t_pad), jnp.int32)
    cls = cls.at[:, 1:2 * L:2].set(tgt.T)
    cls = cls.reshape(N, t_pad // 4, 4).transpose(0, 2, 1).reshape(N, t_pad)

    out = pl.pallas_call(
        functools.partial(_ctc_kernel, s_chunk=s_chunk, t_pad=t_pad,
                          n_sc=n_sc, n_valid_t=T),
        grid=(2, n_sc),
        in_specs=[
            pl.BlockSpec((s_chunk, G, C), lambda g, sc: (sc, g, 0)),
            pl.BlockSpec((G, t_pad), lambda g, sc: (g, 0)),
        ],
        out_specs=pl.BlockSpec((1, G, 128), lambda g, sc: (g, 0, 0)),
        out_shape=jax.ShapeDtypeStruct((2, G, 128), jnp.float32),
        scratch_shapes=[
            pltpu.VMEM((s_chunk, G, t_pad), jnp.float32),
            pltpu.VMEM((G, t_pad), jnp.float32),
        ],
        compiler_params=pltpu.CompilerParams(
            dimension_semantics=("parallel", "arbitrary"),
            vmem_limit_bytes=100 * 1024 * 1024,
        ),
    )(input, cls)

    losses = out.reshape(N, 128)[:, 0]
    return jnp.sum(losses) / N


# speedup vs baseline: 30.1122x; 1.8380x over previous
import functools

import jax
import jax.numpy as jnp
from jax.experimental import pallas as pl
from jax.experimental.pallas import tpu as pltpu

LO = 1e-5
SKIP = -5.0


def _ctc_kernel(in_ref, cls_ref, out_ref, lm_ref, v_ref, *, s_chunk, t_pad,
                n_sc, n_valid_t):
    g = pl.program_id(0)
    sc = pl.program_id(1)
    del g
    G = in_ref.shape[1]
    C = in_ref.shape[2]

    x = in_ref[...]
    xc = jnp.maximum(x, LO)
    ssum = jnp.sum(xc, axis=2, keepdims=True)
    p = (xc / ssum).astype(jnp.bfloat16)
    iota_c = jax.lax.broadcasted_iota(jnp.int32, (C, t_pad), 0)
    for n in range(G):
        cls_row = cls_ref[n, :].reshape(1, t_pad)
        oh = jnp.where(iota_c == cls_row, 1.0, 0.0).astype(jnp.bfloat16)
        pn = p[:, n, :].reshape(s_chunk, C)
        res = jnp.dot(pn, oh, preferred_element_type=jnp.float32)
        lm_ref[:, n, :] = jnp.log(res)

    @pl.when(sc == 0)
    def _():
        lane = jax.lax.broadcasted_iota(jnp.int32, (G, t_pad), 1)
        t_of_lane = 4 * (lane & 127) + (lane >> 7)
        v_ref[...] = SKIP * t_of_lane.astype(jnp.float32)

    lane_i = jax.lax.broadcasted_iota(jnp.int32, (G, 128), 1)
    base = (sc * s_chunk).astype(jnp.float32)
    UNROLL = 8

    def log_add_m(v, w, m):
        d = jnp.abs(v - w)
        return jnp.maximum(v, w) + jnp.log(jnp.exp(-d) + 1.0) + m

    def body(i, carry):
        v0, v1, v2, v3 = carry
        s0 = i * UNROLL
        for u in range(UNROLL):
            m = lm_ref[s0 + u]
            si = base + (s0 + u).astype(jnp.float32)
            w0 = pltpu.roll(v3, 1, axis=1)
            w0 = jnp.where(lane_i == 0, SKIP * si, w0)
            n0 = log_add_m(v0, w0, m[:, 0:128])
            n1 = log_add_m(v1, v0, m[:, 128:256])
            n2 = log_add_m(v2, v1, m[:, 256:384])
            n3 = log_add_m(v3, v2, m[:, 384:512])
            v0, v1, v2, v3 = n0, n1, n2, n3
        return v0, v1, v2, v3

    carry0 = (v_ref[:, 0:128], v_ref[:, 128:256],
              v_ref[:, 256:384], v_ref[:, 384:512])
    v0, v1, v2, v3 = jax.lax.fori_loop(0, s_chunk // UNROLL, body, carry0)
    v_ref[:, 0:128] = v0
    v_ref[:, 128:256] = v1
    v_ref[:, 256:384] = v2
    v_ref[:, 384:512] = v3

    @pl.when(sc == n_sc - 1)
    def _():
        vs = (v0, v1, v2, v3)
        t1, t2 = n_valid_t - 1, n_valid_t - 2
        x1 = vs[t1 % 4][:, t1 // 4:t1 // 4 + 1]
        y1 = vs[t2 % 4][:, t2 // 4:t2 // 4 + 1]
        d = x1 - y1
        la = jnp.where(jnp.abs(d) > 10.0,
                       jnp.maximum(x1, y1),
                       jnp.log(jnp.exp(jnp.clip(d, -20.0, 20.0)) + 1.0) + y1)
        out_ref[...] = jnp.broadcast_to(-la, (G, 128)).reshape(1, G, 128)


@jax.jit
def kernel(input, targets):
    S, N, C = input.shape
    L = targets.shape[0]
    T = 2 * L + 1
    t_pad = 512
    G = N // 2
    s_chunk = 256
    n_sc = S // s_chunk

    tgt = targets.astype(jnp.int32)
    cls = jnp.zeros((N, t_pad), jnp.int32)
    cls = cls.at[:, 1:2 * L:2].set(tgt.T)
    cls = cls.reshape(N, t_pad // 4, 4).transpose(0, 2, 1).reshape(N, t_pad)

    out = pl.pallas_call(
        functools.partial(_ctc_kernel, s_chunk=s_chunk, t_pad=t_pad,
                          n_sc=n_sc, n_valid_t=T),
        grid=(2, n_sc),
        in_specs=[
            pl.BlockSpec((s_chunk, G, C), lambda g, sc: (sc, g, 0)),
            pl.BlockSpec((G, t_pad), lambda g, sc: (g, 0)),
        ],
        out_specs=pl.BlockSpec((1, G, 128), lambda g, sc: (g, 0, 0)),
        out_shape=jax.ShapeDtypeStruct((2, G, 128), jnp.float32),
        scratch_shapes=[
            pltpu.VMEM((s_chunk, G, t_pad), jnp.float32),
            pltpu.VMEM((G, t_pad), jnp.float32),
        ],
        compiler_params=pltpu.CompilerParams(
            dimension_semantics=("parallel", "arbitrary"),
            vmem_limit_bytes=100 * 1024 * 1024,
        ),
    )(input, cls)

    losses = out.reshape(N, 128)[:, 0]
    return jnp.sum(losses) / N
